# Initial kernel scaffold; baseline (speedup 1.0000x reference)
import jax
import jax.numpy as jnp
from jax.experimental import pallas as pl


def kernel(x_src, x_dst, src_idx, dst_idx):
    raise NotImplementedError("write your pallas kernel here")


# speedup vs baseline: 1.1999x; 1.1999x over previous
import functools

import jax
import jax.numpy as jnp
from jax import lax
from jax.experimental import pallas as pl
from jax.experimental.pallas import tpu as pltpu
from jax.experimental.pallas import tpu_sc as plsc

D = 256
N_PAD = 163840
NUM_WORKERS = 32
PER_W = N_PAD // NUM_WORKERS
C = 128
NCHUNK = PER_W // C
L = 16


@functools.partial(
    pl.kernel,
    mesh=plsc.VectorSubcoreMesh(core_axis_name="c", subcore_axis_name="s"),
    out_type=jax.ShapeDtypeStruct((N_PAD,), jnp.float32),
    compiler_params=pltpu.CompilerParams(use_tc_tiling_on_sc=False),
    scratch_types=[
        pltpu.VMEM((C,), jnp.int32),
        pltpu.VMEM((C,), jnp.int32),
        pltpu.VMEM((C, D), jnp.float32),
        pltpu.VMEM((C, D), jnp.float32),
        pltpu.VMEM((C,), jnp.float32),
        pltpu.VMEM((L, 2 * L), jnp.float32),
        pltpu.SemaphoreType.DMA,
        pltpu.SemaphoreType.DMA,
    ],
)
def _score_kernel(xsrc, xdst, sidx_hbm, didx_hbm, out_hbm,
                  sidx_v, didx_v, s_rows, d_rows, out_v, red_v, sem1, sem2):
    wid = lax.axis_index("s") * 2 + lax.axis_index("c")

    lane = lax.iota(jnp.int32, L)
    zeros = jnp.zeros((L,), jnp.float32)
    for j in range(L):
        red_v[j, pl.ds(L, L)] = zeros

    def grp_body(g, carry):
        res = jnp.zeros((L,), jnp.float32)
        for j in range(L):
            e = g * L + j
            accs = [jnp.zeros((L,), jnp.float32) for _ in range(4)]
            for k in range(D // L):
                sv = s_rows[e, pl.ds(k * L, L)]
                dv = d_rows[e, pl.ds(k * L, L)]
                accs[k % 4] = accs[k % 4] + sv * dv
            t = (accs[0] + accs[1]) + (accs[2] + accs[3])
            for h in (8, 4, 2, 1):
                red_v[j, pl.ds(0, L)] = t
                t = t + red_v[j, pl.ds(h, L)]
            res = jnp.where(lane == j, t[0], res)
        out_v[pl.ds(g * L, L)] = res
        return carry

    def chunk_body(c, carry):
        base = wid * PER_W + c * C
        pltpu.sync_copy(sidx_hbm.at[pl.ds(base, C)], sidx_v)
        pltpu.sync_copy(didx_hbm.at[pl.ds(base, C)], didx_v)
        cp1 = pltpu.async_copy(xsrc.at[sidx_v], s_rows, sem1)
        cp2 = pltpu.async_copy(xdst.at[didx_v], d_rows, sem2)
        cp1.wait()
        cp2.wait()
        lax.fori_loop(0, C // L, grp_body, 0)
        pltpu.sync_copy(out_v, out_hbm.at[pl.ds(base, C)])
        return carry

    lax.fori_loop(0, NCHUNK, chunk_body, 0)


def kernel(x_src, x_dst, src_idx, dst_idx):
    E = src_idx.shape[0]
    si = src_idx.astype(jnp.int32)
    di = dst_idx.astype(jnp.int32)
    pad = N_PAD - E
    si = jnp.concatenate([si, jnp.zeros((pad,), jnp.int32)])
    di = jnp.concatenate([di, jnp.zeros((pad,), jnp.int32)])
    out = _score_kernel(x_src, x_dst, si, di)
    return out[:E].reshape(E, 1)


# speedup vs baseline: 1.5407x; 1.2840x over previous
import functools

import jax
import jax.numpy as jnp
from jax import lax
from jax.experimental import pallas as pl
from jax.experimental.pallas import tpu as pltpu
from jax.experimental.pallas import tpu_sc as plsc

D = 256
N_PAD = 163840
NUM_WORKERS = 32
PER_W = N_PAD // NUM_WORKERS
C = 64
NCHUNK = PER_W // C
L = 16


@functools.partial(
    pl.kernel,
    mesh=plsc.VectorSubcoreMesh(core_axis_name="c", subcore_axis_name="s"),
    out_type=jax.ShapeDtypeStruct((N_PAD,), jnp.float32),
    compiler_params=pltpu.CompilerParams(use_tc_tiling_on_sc=False),
    scratch_types=[
        pltpu.VMEM((PER_W,), jnp.int32),
        pltpu.VMEM((PER_W,), jnp.int32),
        pltpu.VMEM((2, C, D), jnp.float32),
        pltpu.VMEM((2, C, D), jnp.float32),
        pltpu.VMEM((2, C), jnp.float32),
        pltpu.VMEM((L, 2 * L), jnp.float32),
        pltpu.SemaphoreType.DMA,
        pltpu.SemaphoreType.DMA,
        pltpu.SemaphoreType.DMA,
        pltpu.SemaphoreType.DMA,
    ],
)
def _score_kernel(xsrc, xdst, sidx_hbm, didx_hbm, out_hbm,
                  sidx_v, didx_v, s_rows, d_rows, out_v, red_v,
                  ssem0, ssem1, dsem0, dsem1):
    wid = lax.axis_index("s") * 2 + lax.axis_index("c")
    base_w = wid * PER_W
    ssems = (ssem0, ssem1)
    dsems = (dsem0, dsem1)

    lane = lax.iota(jnp.int32, L)
    zeros = jnp.zeros((L,), jnp.float32)
    for j in range(L):
        red_v[j, pl.ds(L, L)] = zeros

    pltpu.sync_copy(sidx_hbm.at[pl.ds(base_w, PER_W)], sidx_v)
    pltpu.sync_copy(didx_hbm.at[pl.ds(base_w, PER_W)], didx_v)

    def issue(c, b):
        pltpu.async_copy(xsrc.at[sidx_v.at[pl.ds(c * C, C)]], s_rows.at[b],
                         ssems[b])
        pltpu.async_copy(xdst.at[didx_v.at[pl.ds(c * C, C)]], d_rows.at[b],
                         dsems[b])

    def wait(c, b):
        pltpu.make_async_copy(xsrc.at[sidx_v.at[pl.ds(c * C, C)]],
                              s_rows.at[b], ssems[b]).wait()
        pltpu.make_async_copy(xdst.at[didx_v.at[pl.ds(c * C, C)]],
                              d_rows.at[b], dsems[b]).wait()

    def compute(b):
        def grp_body(g, carry):
            res = jnp.zeros((L,), jnp.float32)
            for j in range(L):
                e = g * L + j
                accs = [jnp.zeros((L,), jnp.float32) for _ in range(4)]
                for k in range(D // L):
                    sv = s_rows[b, e, pl.ds(k * L, L)]
                    dv = d_rows[b, e, pl.ds(k * L, L)]
                    accs[k % 4] = accs[k % 4] + sv * dv
                t = (accs[0] + accs[1]) + (accs[2] + accs[3])
                for h in (8, 4, 2, 1):
                    red_v[j, pl.ds(0, L)] = t
                    t = t + red_v[j, pl.ds(h, L)]
                res = jnp.where(lane == j, t[0], res)
            out_v[b, pl.ds(g * L, L)] = res
            return carry

        lax.fori_loop(0, C // L, grp_body, 0)

    issue(0, 0)

    def pair_body(i, carry):
        for b in (0, 1):
            c = 2 * i + b
            wait(c, b)
            nc = c + 1

            @pl.when(nc < NCHUNK)
            def _():
                issue(nc, 1 - b)

            compute(b)
            pltpu.sync_copy(out_v.at[b], out_hbm.at[pl.ds(base_w + c * C, C)])
        return carry

    lax.fori_loop(0, NCHUNK // 2, pair_body, 0)


def kernel(x_src, x_dst, src_idx, dst_idx):
    E = src_idx.shape[0]
    si = src_idx.astype(jnp.int32)
    di = dst_idx.astype(jnp.int32)
    pad = N_PAD - E
    si = jnp.concatenate([si, jnp.zeros((pad,), jnp.int32)])
    di = jnp.concatenate([di, jnp.zeros((pad,), jnp.int32)])
    out = _score_kernel(x_src, x_dst, si, di)
    return out[:E].reshape(E, 1)


# speedup vs baseline: 1.6241x; 1.0541x over previous
import functools

import jax
import jax.numpy as jnp
from jax import lax
from jax.experimental import pallas as pl
from jax.experimental.pallas import tpu as pltpu
from jax.experimental.pallas import tpu_sc as plsc

D = 256
N_PAD = 163840
NUM_WORKERS = 32
PER_W = N_PAD // NUM_WORKERS
C = 128
NCHUNK = PER_W // C
L = 16


@functools.partial(
    pl.kernel,
    mesh=plsc.VectorSubcoreMesh(core_axis_name="c", subcore_axis_name="s"),
    out_type=jax.ShapeDtypeStruct((N_PAD,), jnp.float32),
    compiler_params=pltpu.CompilerParams(use_tc_tiling_on_sc=False),
    scratch_types=[
        pltpu.VMEM((PER_W,), jnp.int32),
        pltpu.VMEM((PER_W,), jnp.int32),
        pltpu.VMEM((2, C, D // 2), jnp.int32),
        pltpu.VMEM((2, C, D // 2), jnp.int32),
        pltpu.VMEM((2, C), jnp.float32),
        pltpu.VMEM((L, 2 * L), jnp.float32),
        pltpu.SemaphoreType.DMA,
        pltpu.SemaphoreType.DMA,
        pltpu.SemaphoreType.DMA,
        pltpu.SemaphoreType.DMA,
    ],
)
def _score_kernel(xsrc, xdst, sidx_hbm, didx_hbm, out_hbm,
                  sidx_v, didx_v, s_rows, d_rows, out_v, red_v,
                  ssem0, ssem1, dsem0, dsem1):
    wid = lax.axis_index("s") * 2 + lax.axis_index("c")
    base_w = wid * PER_W
    ssems = (ssem0, ssem1)
    dsems = (dsem0, dsem1)

    lane = lax.iota(jnp.int32, L)
    zeros = jnp.zeros((L,), jnp.float32)
    for j in range(L):
        red_v[j, pl.ds(L, L)] = zeros

    pltpu.sync_copy(sidx_hbm.at[pl.ds(base_w, PER_W)], sidx_v)
    pltpu.sync_copy(didx_hbm.at[pl.ds(base_w, PER_W)], didx_v)

    def issue(c, b):
        pltpu.async_copy(xsrc.at[sidx_v.at[pl.ds(c * C, C)]], s_rows.at[b],
                         ssems[b])
        pltpu.async_copy(xdst.at[didx_v.at[pl.ds(c * C, C)]], d_rows.at[b],
                         dsems[b])

    def wait(c, b):
        pltpu.make_async_copy(xsrc.at[sidx_v.at[pl.ds(c * C, C)]],
                              s_rows.at[b], ssems[b]).wait()
        pltpu.make_async_copy(xdst.at[didx_v.at[pl.ds(c * C, C)]],
                              d_rows.at[b], dsems[b]).wait()

    def compute(b):
        def grp_body(g, carry):
            res = jnp.zeros((L,), jnp.float32)
            for j in range(L):
                e = g * L + j
                accs = [jnp.zeros((L,), jnp.float32) for _ in range(4)]
                for k in range(D // (2 * L)):
                    sv = s_rows[b, e, pl.ds(k * L, L)]
                    dv = d_rows[b, e, pl.ds(k * L, L)]
                    s_even = lax.bitcast_convert_type(sv << 16, jnp.float32)
                    d_even = lax.bitcast_convert_type(dv << 16, jnp.float32)
                    s_odd = lax.bitcast_convert_type(sv & jnp.int32(-65536),
                                                     jnp.float32)
                    d_odd = lax.bitcast_convert_type(dv & jnp.int32(-65536),
                                                     jnp.float32)
                    accs[(2 * k) % 4] = accs[(2 * k) % 4] + s_even * d_even
                    accs[(2 * k + 1) % 4] = (accs[(2 * k + 1) % 4]
                                             + s_odd * d_odd)
                t = (accs[0] + accs[1]) + (accs[2] + accs[3])
                for h in (8, 4, 2, 1):
                    red_v[j, pl.ds(0, L)] = t
                    t = t + red_v[j, pl.ds(h, L)]
                res = jnp.where(lane == j, t[0], res)
            out_v[b, pl.ds(g * L, L)] = res
            return carry

        lax.fori_loop(0, C // L, grp_body, 0)

    issue(0, 0)

    def pair_body(i, carry):
        for b in (0, 1):
            c = 2 * i + b
            wait(c, b)
            nc = c + 1

            @pl.when(nc < NCHUNK)
            def _():
                issue(nc, 1 - b)

            compute(b)
            pltpu.sync_copy(out_v.at[b], out_hbm.at[pl.ds(base_w + c * C, C)])
        return carry

    lax.fori_loop(0, NCHUNK // 2, pair_body, 0)


def kernel(x_src, x_dst, src_idx, dst_idx):
    E = src_idx.shape[0]
    si = src_idx.astype(jnp.int32)
    di = dst_idx.astype(jnp.int32)
    pad = N_PAD - E
    si = jnp.concatenate([si, jnp.zeros((pad,), jnp.int32)])
    di = jnp.concatenate([di, jnp.zeros((pad,), jnp.int32)])
    xs_p = lax.bitcast_convert_type(
        x_src.astype(jnp.bfloat16).reshape(-1, D // 2, 2), jnp.int32)
    xd_p = lax.bitcast_convert_type(
        x_dst.astype(jnp.bfloat16).reshape(-1, D // 2, 2), jnp.int32)
    out = _score_kernel(xs_p, xd_p, si, di)
    return out[:E].reshape(E, 1)
